# baseline (speedup 1.0000x reference)
import functools

import jax
import jax.numpy as jnp
from jax import lax
from jax.experimental import pallas as pl
from jax.experimental.pallas import tpu as pltpu
from jax.experimental.pallas import tpu_sc as plsc

_NC = 2
_NS = 16
_L = 16
_S_CHUNK = 8


def _sc_body(seq_len, batch, d_model, x_hbm, pos_hbm, out_hbm, xbuf, posbuf):
    nw = _NC * _NS
    rows_per_w = seq_len // nw
    n_chunks = rows_per_w // _S_CHUNK
    wid = lax.axis_index("s") * _NC + lax.axis_index("c")
    base = wid * rows_per_w

    def chunk_body(g, _):
        s0 = base + g * _S_CHUNK
        pltpu.sync_copy(x_hbm.at[pl.ds(s0, _S_CHUNK)], xbuf)
        pltpu.sync_copy(pos_hbm.at[pl.ds(s0, _S_CHUNK)], posbuf)

        def d_body(i, _):
            off = i * _L
            for j in range(_S_CHUNK):
                p = posbuf[j, pl.ds(off, _L)]
                for b in range(batch):
                    xbuf[j, b, pl.ds(off, _L)] = xbuf[j, b, pl.ds(off, _L)] + p
            return _

        lax.fori_loop(0, d_model // _L, d_body, None)
        pltpu.sync_copy(xbuf, out_hbm.at[pl.ds(s0, _S_CHUNK)])
        return _

    lax.fori_loop(0, n_chunks, chunk_body, None)


def kernel(x, pos_table):
    seq_len, batch, d_model = x.shape
    mesh = plsc.VectorSubcoreMesh(
        core_axis_name="c", subcore_axis_name="s",
        num_cores=_NC, num_subcores=_NS,
    )
    body = functools.partial(_sc_body, seq_len, batch, d_model)
    return pl.kernel(
        body,
        out_type=jax.ShapeDtypeStruct((seq_len, batch, d_model), x.dtype),
        mesh=mesh,
        scratch_types=[
            pltpu.VMEM((_S_CHUNK, batch, d_model), jnp.float32),
            pltpu.VMEM((_S_CHUNK, d_model), jnp.float32),
        ],
    )(x, pos_table[:seq_len])


# speedup vs baseline: 1.4859x; 1.4859x over previous
import functools

import jax
import jax.numpy as jnp
from jax import lax
from jax.experimental import pallas as pl
from jax.experimental.pallas import tpu as pltpu
from jax.experimental.pallas import tpu_sc as plsc

_NC = 2
_NS = 16
_L = 16
_S_CHUNK = 8
_NBUF = 2


def _sc_body(seq_len, batch, d_model,
             x_hbm, pos_hbm, out_hbm, xbuf, posbuf, sem_in, sem_out):
    nw = _NC * _NS
    rows_per_w = seq_len // nw
    n_chunks = rows_per_w // _S_CHUNK
    wid = lax.axis_index("s") * _NC + lax.axis_index("c")
    base = wid * rows_per_w

    def start_loads(g, slot):
        s0 = base + g * _S_CHUNK
        dx = pltpu.async_copy(x_hbm.at[pl.ds(s0, _S_CHUNK)], xbuf.at[slot],
                              sem_in.at[slot])
        dp = pltpu.async_copy(pos_hbm.at[pl.ds(s0, _S_CHUNK)], posbuf.at[slot],
                              sem_in.at[slot])
        return dx, dp

    def compute(slot):
        def d_body(i, _):
            off = i * _L
            for j in range(_S_CHUNK):
                p = posbuf[slot, j, pl.ds(off, _L)]
                for b in range(batch):
                    xbuf[slot, j, b, pl.ds(off, _L)] = (
                        xbuf[slot, j, b, pl.ds(off, _L)] + p)
            return _
        lax.fori_loop(0, d_model // _L, d_body, None)

    stores = [None] * _NBUF
    loads = [None] * _NBUF
    loads[0] = start_loads(0, 0)
    for g in range(n_chunks):
        slot = g % _NBUF
        nslot = (g + 1) % _NBUF
        if g + 1 < n_chunks:
            if stores[nslot] is not None:
                stores[nslot].wait()
                stores[nslot] = None
            loads[nslot] = start_loads(g + 1, nslot)
        dx, dp = loads[slot]
        dx.wait()
        dp.wait()
        compute(slot)
        s0 = base + g * _S_CHUNK
        stores[slot] = pltpu.async_copy(
            xbuf.at[slot], out_hbm.at[pl.ds(s0, _S_CHUNK)], sem_out.at[slot])
    for d in stores:
        if d is not None:
            d.wait()


def kernel(x, pos_table):
    seq_len, batch, d_model = x.shape
    mesh = plsc.VectorSubcoreMesh(
        core_axis_name="c", subcore_axis_name="s",
        num_cores=_NC, num_subcores=_NS,
    )
    body = functools.partial(_sc_body, seq_len, batch, d_model)
    return pl.kernel(
        body,
        out_type=jax.ShapeDtypeStruct((seq_len, batch, d_model), x.dtype),
        mesh=mesh,
        scratch_types=[
            pltpu.VMEM((_NBUF, _S_CHUNK, batch, d_model), jnp.float32),
            pltpu.VMEM((_NBUF, _S_CHUNK, d_model), jnp.float32),
            pltpu.SemaphoreType.DMA((_NBUF,)),
            pltpu.SemaphoreType.DMA((_NBUF,)),
        ],
    )(x, pos_table[:seq_len])


# speedup vs baseline: 1.9943x; 1.3421x over previous
import functools

import jax
import jax.numpy as jnp
from jax import lax
from jax.experimental import pallas as pl
from jax.experimental.pallas import tpu as pltpu
from jax.experimental.pallas import tpu_sc as plsc

_NC = 2
_NS = 16
_L = 16
_S_CHUNK = 8
_NBUF = 3


def _sc_body(seq_len, batch, d_model,
             x_hbm, pos_hbm, out_hbm, xbuf, posbuf, sem_in, sem_out):
    nw = _NC * _NS
    rows_per_w = seq_len // nw
    n_chunks = rows_per_w // _S_CHUNK
    wid = lax.axis_index("s") * _NC + lax.axis_index("c")
    base = wid * rows_per_w

    def start_loads(g, slot):
        s0 = base + g * _S_CHUNK
        dx = pltpu.async_copy(x_hbm.at[pl.ds(s0, _S_CHUNK)], xbuf.at[slot],
                              sem_in.at[slot])
        dp = pltpu.async_copy(pos_hbm.at[pl.ds(s0, _S_CHUNK)], posbuf.at[slot],
                              sem_in.at[slot])
        return dx, dp

    def compute(slot):
        def d_body(i, _):
            off = i * _L
            for j in range(_S_CHUNK):
                p = posbuf[slot, j, pl.ds(off, _L)]
                for b in range(batch):
                    plsc.addupdate(xbuf.at[slot, j, b, pl.ds(off, _L)], p)
            return _
        lax.fori_loop(0, d_model // _L, d_body, None)

    stores = [None] * _NBUF
    loads = [None] * _NBUF
    loads[0] = start_loads(0, 0)
    for g in range(n_chunks):
        slot = g % _NBUF
        nslot = (g + 1) % _NBUF
        if g + 1 < n_chunks:
            if stores[nslot] is not None:
                stores[nslot].wait()
                stores[nslot] = None
            loads[nslot] = start_loads(g + 1, nslot)
        dx, dp = loads[slot]
        dx.wait()
        dp.wait()
        compute(slot)
        s0 = base + g * _S_CHUNK
        stores[slot] = pltpu.async_copy(
            xbuf.at[slot], out_hbm.at[pl.ds(s0, _S_CHUNK)], sem_out.at[slot])
    for d in stores:
        if d is not None:
            d.wait()


def kernel(x, pos_table):
    seq_len, batch, d_model = x.shape
    mesh = plsc.VectorSubcoreMesh(
        core_axis_name="c", subcore_axis_name="s",
        num_cores=_NC, num_subcores=_NS,
    )
    body = functools.partial(_sc_body, seq_len, batch, d_model)
    return pl.kernel(
        body,
        out_type=jax.ShapeDtypeStruct((seq_len, batch, d_model), x.dtype),
        mesh=mesh,
        scratch_types=[
            pltpu.VMEM((_NBUF, _S_CHUNK, batch, d_model), jnp.float32),
            pltpu.VMEM((_NBUF, _S_CHUNK, d_model), jnp.float32),
            pltpu.SemaphoreType.DMA((_NBUF,)),
            pltpu.SemaphoreType.DMA((_NBUF,)),
        ],
    )(x, pos_table[:seq_len])
